# Initial kernel scaffold; baseline (speedup 1.0000x reference)
import jax
import jax.numpy as jnp
from jax.experimental import pallas as pl


def kernel(obs, emb_table, W, b):
    raise NotImplementedError("write your pallas kernel here")


# speedup vs baseline: 2.3583x; 2.3583x over previous
import jax
import jax.numpy as jnp
from jax.experimental import pallas as pl
from jax.experimental.pallas import tpu as pltpu

_BATCH = 32
_OBS_DIM = 12
_VOCAB = 10
_EMBED = 16
_OUT = 128
_ROWS = _BATCH * _OBS_DIM
_FAN = _EMBED * _OBS_DIM


def _tc_body(obs_rep_ref, embt_ref, w_ref, b_ref, o_ref):
    obs_rep = obs_rep_ref[...]
    embt = embt_ref[...]
    flat = jnp.zeros((_ROWS, _FAN), jnp.float32)
    for v in range(_VOCAB):
        flat = flat + jnp.where(obs_rep == v, embt[v][None, :], 0.0)
    out = jax.lax.dot_general(
        flat, w_ref[...], (((1,), (0,)), ((), ())),
        preferred_element_type=jnp.float32)
    o_ref[...] = out + b_ref[...]


def kernel(obs, emb_table, W, b):
    obs2 = obs.reshape(_ROWS, _OBS_DIM).astype(jnp.int32)
    obs_rep = jnp.repeat(obs2, _EMBED, axis=1)
    embt = jnp.tile(emb_table, (1, _OBS_DIM))
    out = pl.pallas_call(
        _tc_body,
        out_shape=jax.ShapeDtypeStruct((_ROWS, _OUT), jnp.float32),
    )(obs_rep, embt, W, b.reshape(1, _OUT))
    return out.reshape(_BATCH, _OBS_DIM, _OUT)
